# baseline (speedup 1.0000x reference)
import jax
import jax.numpy as jnp
import numpy as np
from jax.experimental import pallas as pl
from jax.experimental.pallas import tpu as pltpu

_V = 100000
_D = 1024
_NSPLIT = 4
_CHUNK = 1024
_STEP = _NSPLIT * _CHUNK
_NV = -(-_V // _STEP)
_TOPK = 40
_TOPP = 0.9
_NEG = np.float32(-np.inf)


def _body(s_ref, tok_ref, seg_ref, m_ref, *rest):
    w_refs = rest[:_NSPLIT]
    g_ref = rest[_NSPLIT]
    out_ref = rest[_NSPLIT + 1]
    L_ref = rest[_NSPLIT + 2]
    i = pl.program_id(0)
    h = (tok_ref[0] + seg_ref[0]) * m_ref[...]
    hrep = jnp.broadcast_to(h, (8, _D))
    col = jax.lax.broadcasted_iota(jnp.int32, (1, _CHUNK), 1)
    for k in range(_NSPLIT):
        chunk = jax.lax.dot_general(
            hrep, w_refs[k][...], (((1,), (0,)), ((), ())),
            preferred_element_type=jnp.float32)
        valid = (i * _STEP + k * _CHUNK + col) < _V
        L_ref[pl.ds(i, 1), k * _CHUNK:(k + 1) * _CHUNK] = jnp.where(
            valid, chunk[0:1, :], _NEG)

    @pl.when(i == _NV - 1)
    def _finalize():
        lane = jax.lax.broadcasted_iota(jnp.int32, (1, 128), 1)

        def step(t, carry):
            m_prev, vals, counts = carry
            Lv = L_ref[...]
            m = jnp.max(jnp.where(Lv < m_prev, Lv, _NEG))
            c = jnp.sum(jnp.where(Lv == m, 1.0, 0.0).astype(jnp.float32))
            vals = jnp.where(lane == t, m, vals)
            counts = jnp.where(lane == t, c, counts)
            return m, vals, counts

        _, vals, counts = jax.lax.fori_loop(
            0, _TOPK, step,
            (np.float32(np.inf),
             jnp.full((1, 128), _NEG, jnp.float32),
             jnp.zeros((1, 128), jnp.float32)))

        mtop = jnp.max(vals)
        tri = (jax.lax.broadcasted_iota(jnp.int32, (128, 128), 0)
               <= jax.lax.broadcasted_iota(jnp.int32, (128, 128), 1)
               ).astype(jnp.float32)
        cum_counts = jax.lax.dot_general(
            counts, tri, (((1,), (0,)), ((), ())),
            precision=jax.lax.Precision.HIGHEST)
        excl = cum_counts - counts
        kept = jnp.logical_and(excl < np.float32(_TOPK), counts > 0.0)
        p_raw = jnp.exp(vals - mtop)
        w = jnp.where(kept, counts * p_raw, 0.0)
        Z = jnp.sum(w)
        p = p_raw / Z
        Cw = jax.lax.dot_general(
            w, tri, (((1,), (0,)), ((), ())),
            precision=jax.lax.Precision.HIGHEST) / Z
        cond = jnp.logical_and(kept, (Cw - p) > np.float32(_TOPP))
        kth = jnp.min(jnp.where(kept, vals, np.float32(np.inf)))
        cutoff = jnp.maximum(jnp.max(jnp.where(cond, vals, _NEG)), kth)

        L = L_ref[...]
        Y = jnp.where(L >= cutoff, L + g_ref[...], _NEG)
        ymax = jnp.max(Y)
        ridx = jax.lax.broadcasted_iota(jnp.int32, (_NV, _STEP), 0)
        cidx = jax.lax.broadcasted_iota(jnp.int32, (_NV, _STEP), 1)
        fidx = ridx * _STEP + cidx
        widx = jnp.min(jnp.where(Y == ymax, fidx, np.int32(2**31 - 1)))
        out_ref[...] = jnp.broadcast_to(widx, (1, 1))


def kernel(src_tensor, seg_tensor, mask, tok_emb, seg_emb, W):
    idxs = jnp.concatenate(
        [src_tensor[0, -1:], seg_tensor[0, -1:]]).astype(jnp.int32)
    mlast = mask[:, -1:]
    g = jax.random.gumbel(jax.random.key(42), (1, _V), jnp.float32)
    g = jnp.pad(g, ((0, 0), (0, _NV * _STEP - _V))).reshape(_NV, _STEP)

    nblocks = -(-_V // _CHUNK)

    def _w_spec(k):
        return pl.BlockSpec(
            (_D, _CHUNK),
            lambda i, s, k=k: (0, jnp.minimum(i * _NSPLIT + k, nblocks - 1)))

    grid_spec = pltpu.PrefetchScalarGridSpec(
        num_scalar_prefetch=1,
        grid=(_NV,),
        in_specs=[
            pl.BlockSpec((1, 1, _D), lambda i, s: (s[0], 0, 0)),
            pl.BlockSpec((1, 1, _D), lambda i, s: (s[1], 0, 0)),
            pl.BlockSpec((1, 1), lambda i, s: (0, 0)),
            *[_w_spec(k) for k in range(_NSPLIT)],
            pl.BlockSpec((_NV, _STEP), lambda i, s: (0, 0)),
        ],
        out_specs=pl.BlockSpec((1, 1), lambda i, s: (0, 0)),
        scratch_shapes=[pltpu.VMEM((_NV, _STEP), jnp.float32)],
    )
    out = pl.pallas_call(
        _body,
        grid_spec=grid_spec,
        out_shape=jax.ShapeDtypeStruct((1, 1), jnp.int32),
    )(idxs, tok_emb.reshape(_V, 1, _D), seg_emb.reshape(2, 1, _D),
      mlast, *([W] * _NSPLIT), g)
    return out


# speedup vs baseline: 1.0192x; 1.0192x over previous
import jax
import jax.numpy as jnp
import numpy as np
from jax.experimental import pallas as pl
from jax.experimental.pallas import tpu as pltpu

_V = 100000
_D = 1024
_BD = 32
_NK = _D // _BD
_PR = 8
_PC = _V // _PR
_TOPK = 40
_TOPP = 0.9
_NEG = np.float32(-np.inf)


def _body(s_ref, tok_ref, seg_ref, m_ref, w_ref, g_ref, out_ref,
          h_ref, acc_ref, L_ref):
    i = pl.program_id(0)

    @pl.when(i == 0)
    def _init_h():
        h = (tok_ref[0] + seg_ref[0]) * m_ref[...]
        h_ref[...] = jnp.transpose(h, (1, 0))

    hs = h_ref[pl.ds(i * _BD, _BD), :]
    part = jax.lax.dot_general(
        hs, w_ref[...], (((0,), (0,)), ((), ())),
        preferred_element_type=jnp.float32)

    @pl.when(i == 0)
    def _first():
        acc_ref[...] = part

    @pl.when(i > 0)
    def _rest():
        acc_ref[...] = acc_ref[...] + part

    @pl.when(i == _NK - 1)
    def _finalize():
        for s in range(_PR):
            L_ref[s:s + 1, :] = acc_ref[:, s * _PC:(s + 1) * _PC]

        lane = jax.lax.broadcasted_iota(jnp.int32, (1, 128), 1)

        def step(t, carry):
            m_prev, vals, counts = carry
            Lv = L_ref[...]
            m = jnp.max(jnp.where(Lv < m_prev, Lv, _NEG))
            c = jnp.sum(jnp.where(Lv == m, 1.0, 0.0).astype(jnp.float32))
            vals = jnp.where(lane == t, m, vals)
            counts = jnp.where(lane == t, c, counts)
            return m, vals, counts

        _, vals, counts = jax.lax.fori_loop(
            0, _TOPK, step,
            (np.float32(np.inf),
             jnp.full((1, 128), _NEG, jnp.float32),
             jnp.zeros((1, 128), jnp.float32)))

        mtop = jnp.max(vals)
        tri = (jax.lax.broadcasted_iota(jnp.int32, (128, 128), 0)
               <= jax.lax.broadcasted_iota(jnp.int32, (128, 128), 1)
               ).astype(jnp.float32)
        cum_counts = jax.lax.dot_general(
            counts, tri, (((1,), (0,)), ((), ())),
            precision=jax.lax.Precision.HIGHEST)
        excl = cum_counts - counts
        kept = jnp.logical_and(excl < np.float32(_TOPK), counts > 0.0)
        p_raw = jnp.exp(vals - mtop)
        w = jnp.where(kept, counts * p_raw, 0.0)
        Z = jnp.sum(w)
        p = p_raw / Z
        Cw = jax.lax.dot_general(
            w, tri, (((1,), (0,)), ((), ())),
            precision=jax.lax.Precision.HIGHEST) / Z
        cond = jnp.logical_and(kept, (Cw - p) > np.float32(_TOPP))
        kth = jnp.min(jnp.where(kept, vals, np.float32(np.inf)))
        cutoff = jnp.maximum(jnp.max(jnp.where(cond, vals, _NEG)), kth)

        L = L_ref[...]
        Y = jnp.where(L >= cutoff, L + g_ref[...], _NEG)
        ymax = jnp.max(Y)
        ridx = jax.lax.broadcasted_iota(jnp.int32, (_PR, _PC), 0)
        cidx = jax.lax.broadcasted_iota(jnp.int32, (_PR, _PC), 1)
        fidx = ridx * _PC + cidx
        widx = jnp.min(jnp.where(Y == ymax, fidx, np.int32(2**31 - 1)))
        out_ref[...] = jnp.broadcast_to(widx, (1, 1))


def kernel(src_tensor, seg_tensor, mask, tok_emb, seg_emb, W):
    idxs = jnp.concatenate(
        [src_tensor[0, -1:], seg_tensor[0, -1:]]).astype(jnp.int32)
    mlast = mask[:, -1:]
    g = jax.random.gumbel(jax.random.key(42), (1, _V), jnp.float32)
    g = g.reshape(_PR, _PC)

    grid_spec = pltpu.PrefetchScalarGridSpec(
        num_scalar_prefetch=1,
        grid=(_NK,),
        in_specs=[
            pl.BlockSpec((1, 1, _D), lambda i, s: (s[0], 0, 0)),
            pl.BlockSpec((1, 1, _D), lambda i, s: (s[1], 0, 0)),
            pl.BlockSpec((1, 1), lambda i, s: (0, 0)),
            pl.BlockSpec((_BD, _V), lambda i, s: (i, 0)),
            pl.BlockSpec((_PR, _PC), lambda i, s: (0, 0)),
        ],
        out_specs=pl.BlockSpec((1, 1), lambda i, s: (0, 0)),
        scratch_shapes=[
            pltpu.VMEM((_D, 1), jnp.float32),
            pltpu.VMEM((1, _V), jnp.float32),
            pltpu.VMEM((_PR, _PC), jnp.float32),
        ],
    )
    out = pl.pallas_call(
        _body,
        grid_spec=grid_spec,
        out_shape=jax.ShapeDtypeStruct((1, 1), jnp.int32),
    )(idxs, tok_emb.reshape(_V, 1, _D), seg_emb.reshape(2, 1, _D),
      mlast, W, g)
    return out
